# baseline (speedup 1.0000x reference)
import functools

import jax
import jax.numpy as jnp
from jax import lax
from jax.experimental import pallas as pl
from jax.experimental.pallas import tpu as pltpu
from jax.experimental.pallas import tpu_sc as plsc

_NC = 2
_NS = 16
_K = 128


def _make_sc_agg(n_nodes, e_pad, dh, edge_split):
    n_workers = _NC * _NS if edge_split else _NS
    assert e_pad % (n_workers * 2 * _K) == 0
    e_w = e_pad // n_workers
    nch = e_w // _K
    rows_tile = -(-(n_nodes // _NS) // 8) * 8
    rows_last = n_nodes - (_NS - 1) * rows_tile
    assert 0 < rows_last <= rows_tile
    assert dh % 16 == 0

    mesh = plsc.VectorSubcoreMesh(
        core_axis_name="c", subcore_axis_name="s",
        num_cores=_NC, num_subcores=_NS)

    scratch = [
        pltpu.VMEM((_K,), jnp.int32), pltpu.VMEM((_K,), jnp.int32),
        pltpu.VMEM((_K,), jnp.int32), pltpu.VMEM((_K,), jnp.int32),
        pltpu.VMEM((_K, dh), jnp.float32), pltpu.VMEM((_K, dh), jnp.float32),
        pltpu.VMEM_SHARED((n_nodes + 8, dh), jnp.float32),
        pltpu.SemaphoreType.DMA, pltpu.SemaphoreType.DMA,
    ]

    @functools.partial(
        pl.kernel,
        out_type=jax.ShapeDtypeStruct((_NC, n_nodes, dh), jnp.float32),
        mesh=mesh,
        scratch_types=scratch,
        compiler_params=pltpu.CompilerParams(use_tc_tiling_on_sc=False),
    )
    def agg_kernel(t_hbm, src_hbm, dst_hbm, z_hbm, out_hbm,
                   si0, si1, di0, di1, rw0, rw1, acc, sem0, sem1):
        c = lax.axis_index("c")
        s = lax.axis_index("s")
        if edge_split:
            tbl = t_hbm
            base = (c * _NS + s) * e_w
        else:
            tbl = t_hbm.at[c]
            base = s * e_w

        r0 = s * rows_tile

        @pl.when(s < _NS - 1)
        def _():
            pltpu.sync_copy(z_hbm.at[pl.ds(r0, rows_tile)],
                            acc.at[pl.ds(r0, rows_tile)])

        @pl.when(s == _NS - 1)
        def _():
            pltpu.sync_copy(z_hbm.at[pl.ds(r0, rows_last)],
                            acc.at[pl.ds(r0, rows_last)])

        plsc.subcore_barrier()

        bufs = ((si0, di0, rw0, sem0), (si1, di1, rw1, sem1))

        pltpu.sync_copy(src_hbm.at[pl.ds(base, _K)], si0)
        pltpu.sync_copy(dst_hbm.at[pl.ds(base, _K)], di0)
        pltpu.async_copy(tbl.at[si0], rw0, sem0)

        def stage(g, b):
            si, di, rw, sem = bufs[b]
            nsi, ndi, nrw, nsem = bufs[1 - b]
            nxt = g + 1

            @pl.when(nxt < nch)
            def _():
                off = base + nxt * _K
                pltpu.sync_copy(src_hbm.at[pl.ds(off, _K)], nsi)
                pltpu.sync_copy(dst_hbm.at[pl.ds(off, _K)], ndi)

            pltpu.make_async_copy(tbl.at[si], rw, sem).wait()

            @pl.when(nxt < nch)
            def _():
                pltpu.async_copy(tbl.at[nsi], nrw, nsem)

            pltpu.sync_copy(rw, acc.at[di], add=True)

        @pl.loop(0, nch, step=2)
        def _(g):
            stage(g, 0)
            stage(g + 1, 1)

        plsc.subcore_barrier()

        @pl.when(s < _NS - 1)
        def _():
            pltpu.sync_copy(acc.at[pl.ds(r0, rows_tile)],
                            out_hbm.at[c, pl.ds(r0, rows_tile)])

        @pl.when(s == _NS - 1)
        def _():
            pltpu.sync_copy(acc.at[pl.ds(r0, rows_last)],
                            out_hbm.at[c, pl.ds(r0, rows_last)])

    return agg_kernel


_BM = 400


def _dot(a, b):
    return jnp.dot(a, b, preferred_element_type=jnp.float32)


def _layer1_tc(x, agg1, W1s, W1np, b1r, n_nodes, dh_in):
    grid = n_nodes // _BM

    def body(x_ref, a_ref, ws_ref, wn_ref, b_ref, h_ref, rdeg_ref):
        asum = a_ref[0] + a_ref[1]
        deg = asum[:, 128:129]
        r = 1.0 / jnp.maximum(deg, 1.0)
        neigh = _dot(asum, wn_ref[...]) * r
        h = jax.nn.relu(_dot(x_ref[...], ws_ref[...]) + neigh + b_ref[...])
        h_ref[0] = h[:, :128]
        h_ref[1] = h[:, 128:]
        rdeg_ref[...] = r

    return pl.pallas_call(
        body,
        grid=(grid,),
        in_specs=[
            pl.BlockSpec((_BM, dh_in), lambda m: (m, 0)),
            pl.BlockSpec((2, _BM, 144), lambda m: (0, m, 0)),
            pl.BlockSpec((dh_in, 256), lambda m: (0, 0)),
            pl.BlockSpec((144, 256), lambda m: (0, 0)),
            pl.BlockSpec((1, 256), lambda m: (0, 0)),
        ],
        out_specs=[
            pl.BlockSpec((2, _BM, 128), lambda m: (0, m, 0)),
            pl.BlockSpec((_BM, 1), lambda m: (m, 0)),
        ],
        out_shape=[
            jax.ShapeDtypeStruct((2, n_nodes, 128), jnp.float32),
            jax.ShapeDtypeStruct((n_nodes, 1), jnp.float32),
        ],
    )(x, agg1, W1s, W1np, b1r)


def _layer2_tc(h1s, agg2, rdeg, W2sa, W2sb, W2na, W2nb, b2r, W3np, n_nodes):
    grid = n_nodes // _BM

    def body(h_ref, a_ref, r_ref, wsa, wsb, wna, wnb, b_ref, w3n, h2_ref, t3_ref):
        r = r_ref[...]
        neigh = (_dot(a_ref[0], wna[...]) + _dot(a_ref[1], wnb[...])) * r
        h2 = jax.nn.relu(_dot(h_ref[0], wsa[...]) + _dot(h_ref[1], wsb[...])
                         + neigh + b_ref[...])
        t3_ref[...] = _dot(h2, w3n[...])
        h2_ref[0] = h2[:, :128]
        h2_ref[1] = h2[:, 128:]

    return pl.pallas_call(
        body,
        grid=(grid,),
        in_specs=[
            pl.BlockSpec((2, _BM, 128), lambda m: (0, m, 0)),
            pl.BlockSpec((2, _BM, 128), lambda m: (0, m, 0)),
            pl.BlockSpec((_BM, 1), lambda m: (m, 0)),
            pl.BlockSpec((128, 256), lambda m: (0, 0)),
            pl.BlockSpec((128, 256), lambda m: (0, 0)),
            pl.BlockSpec((128, 256), lambda m: (0, 0)),
            pl.BlockSpec((128, 256), lambda m: (0, 0)),
            pl.BlockSpec((1, 256), lambda m: (0, 0)),
            pl.BlockSpec((256, 64), lambda m: (0, 0)),
        ],
        out_specs=[
            pl.BlockSpec((2, _BM, 128), lambda m: (0, m, 0)),
            pl.BlockSpec((_BM, 64), lambda m: (m, 0)),
        ],
        out_shape=[
            jax.ShapeDtypeStruct((2, n_nodes, 128), jnp.float32),
            jax.ShapeDtypeStruct((n_nodes, 64), jnp.float32),
        ],
    )(h1s, agg2, rdeg, W2sa, W2sb, W2na, W2nb, b2r, W3np)


def _layer3_tc(h2s, agg3, rdeg, W3sa, W3sb, b3p, n_nodes):
    grid = n_nodes // _BM

    def body(h_ref, a_ref, r_ref, wsa, wsb, b_ref, o_ref):
        neigh = (a_ref[0] + a_ref[1]) * r_ref[...]
        o_ref[...] = (_dot(h_ref[0], wsa[...]) + _dot(h_ref[1], wsb[...])
                      + neigh + b_ref[...])

    return pl.pallas_call(
        body,
        grid=(grid,),
        in_specs=[
            pl.BlockSpec((2, _BM, 128), lambda m: (0, m, 0)),
            pl.BlockSpec((2, _BM, 64), lambda m: (0, m, 0)),
            pl.BlockSpec((_BM, 1), lambda m: (m, 0)),
            pl.BlockSpec((128, 64), lambda m: (0, 0)),
            pl.BlockSpec((128, 64), lambda m: (0, 0)),
            pl.BlockSpec((1, 64), lambda m: (0, 0)),
        ],
        out_specs=pl.BlockSpec((_BM, 64), lambda m: (m, 0)),
        out_shape=jax.ShapeDtypeStruct((n_nodes, 64), jnp.float32),
    )(h2s, agg3, rdeg, W3sa, W3sb, b3p)


def kernel(x, edge_index, W1_self, W1_neigh, b1, W2_self, W2_neigh, b2,
           W3_self, W3_neigh, b3):
    n_nodes, d_in = x.shape
    n_edges = edge_index.shape[1]
    d_out = W3_self.shape[1]

    unit = _NC * _NS * 2 * _K
    e_pad = -(-n_edges // unit) * unit
    src = jnp.concatenate(
        [edge_index[0], jnp.zeros((e_pad - n_edges,), jnp.int32)])
    dst = jnp.concatenate(
        [edge_index[1],
         jnp.full((e_pad - n_edges,), n_nodes, jnp.int32)])

    ones = jnp.ones((n_nodes, 1), jnp.float32)
    zpad = jnp.zeros((n_nodes, 15), jnp.float32)
    xaug = jnp.concatenate([x, ones, zpad], axis=1)

    z144 = jnp.zeros((n_nodes, 144), jnp.float32)
    z128 = jnp.zeros((n_nodes, 128), jnp.float32)
    z64 = jnp.zeros((n_nodes, 64), jnp.float32)

    W1np = jnp.concatenate([W1_neigh, jnp.zeros((16, 256), jnp.float32)],
                           axis=0)
    b1r = b1.reshape(1, -1)
    W2sa, W2sb = W2_self[:128], W2_self[128:]
    W2na, W2nb = W2_neigh[:128], W2_neigh[128:]
    b2r = b2.reshape(1, -1)
    cpad = jnp.zeros((256, 64 - d_out), jnp.float32)
    W3np = jnp.concatenate([W3_neigh, cpad], axis=1)
    W3sp = jnp.concatenate([W3_self, cpad], axis=1)
    W3sa, W3sb = W3sp[:128], W3sp[128:]
    b3p = jnp.concatenate([b3, jnp.zeros((64 - d_out,), jnp.float32)]
                          ).reshape(1, -1)

    agg1 = _make_sc_agg(n_nodes, e_pad, 144, True)(xaug, src, dst, z144)
    h1s, rdeg = _layer1_tc(x, agg1, W1_self, W1np, b1r, n_nodes, d_in)
    agg2 = _make_sc_agg(n_nodes, e_pad, 128, False)(h1s, src, dst, z128)
    h2s, t3 = _layer2_tc(h1s, agg2, rdeg, W2sa, W2sb, W2na, W2nb, b2r,
                         W3np, n_nodes)
    agg3 = _make_sc_agg(n_nodes, e_pad, 64, True)(t3, src, dst, z64)
    outp = _layer3_tc(h2s, agg3, rdeg, W3sa, W3sb, b3p, n_nodes)
    return outp[:, :d_out]


# speedup vs baseline: 2.3386x; 2.3386x over previous
import functools

import jax
import jax.numpy as jnp
from jax import lax
from jax.experimental import pallas as pl
from jax.experimental.pallas import tpu as pltpu
from jax.experimental.pallas import tpu_sc as plsc

_NC = 2
_NS = 16
_K = 128
_TRASH = 256


def _make_sc_agg(n_nodes, e_pad, dh, edge_split):
    n_workers = _NC * _NS if edge_split else _NS
    assert e_pad % (n_workers * 2 * _K) == 0
    trash = _TRASH
    e_w = e_pad // n_workers
    nch = e_w // _K
    rows_tile = -(-(n_nodes // _NS) // 8) * 8
    rows_last = n_nodes - (_NS - 1) * rows_tile
    assert 0 < rows_last <= rows_tile
    assert dh % 16 == 0

    mesh = plsc.VectorSubcoreMesh(
        core_axis_name="c", subcore_axis_name="s",
        num_cores=_NC, num_subcores=_NS)

    scratch = [
        pltpu.VMEM((_K,), jnp.int32), pltpu.VMEM((_K,), jnp.int32),
        pltpu.VMEM((_K,), jnp.int32), pltpu.VMEM((_K,), jnp.int32),
        pltpu.VMEM((_K, dh), jnp.float32), pltpu.VMEM((_K, dh), jnp.float32),
        pltpu.VMEM_SHARED((n_nodes + trash, dh), jnp.float32),
        pltpu.SemaphoreType.DMA, pltpu.SemaphoreType.DMA,
    ]

    @functools.partial(
        pl.kernel,
        out_type=jax.ShapeDtypeStruct((_NC, n_nodes, dh), jnp.float32),
        mesh=mesh,
        scratch_types=scratch,
        compiler_params=pltpu.CompilerParams(use_tc_tiling_on_sc=False),
    )
    def agg_kernel(t_hbm, src_hbm, dst_hbm, z_hbm, out_hbm,
                   si0, si1, di0, di1, rw0, rw1, acc, sem0, sem1):
        c = lax.axis_index("c")
        s = lax.axis_index("s")
        if edge_split:
            tbl = t_hbm
            base = (c * _NS + s) * e_w
        else:
            tbl = t_hbm.at[c]
            base = s * e_w

        r0 = s * rows_tile

        @pl.when(s < _NS - 1)
        def _():
            pltpu.sync_copy(z_hbm.at[pl.ds(r0, rows_tile)],
                            acc.at[pl.ds(r0, rows_tile)])

        @pl.when(s == _NS - 1)
        def _():
            pltpu.sync_copy(z_hbm.at[pl.ds(r0, rows_last)],
                            acc.at[pl.ds(r0, rows_last)])

        plsc.subcore_barrier()

        bufs = ((si0, di0, rw0, sem0), (si1, di1, rw1, sem1))

        pltpu.sync_copy(src_hbm.at[pl.ds(base, _K)], si0)
        pltpu.sync_copy(dst_hbm.at[pl.ds(base, _K)], di0)
        pltpu.async_copy(tbl.at[si0], rw0, sem0)

        def stage(g, b):
            si, di, rw, sem = bufs[b]
            nsi, ndi, nrw, nsem = bufs[1 - b]
            nxt = g + 1

            @pl.when(nxt < nch)
            def _():
                off = base + nxt * _K
                pltpu.sync_copy(src_hbm.at[pl.ds(off, _K)], nsi)
                pltpu.sync_copy(dst_hbm.at[pl.ds(off, _K)], ndi)

            pltpu.make_async_copy(tbl.at[si], rw, sem).wait()

            @pl.when(nxt < nch)
            def _():
                pltpu.async_copy(tbl.at[nsi], nrw, nsem)

            pltpu.sync_copy(rw, acc.at[di], add=True)

        @pl.loop(0, nch, step=2)
        def _(g):
            stage(g, 0)
            stage(g + 1, 1)

        plsc.subcore_barrier()

        @pl.when(s < _NS - 1)
        def _():
            pltpu.sync_copy(acc.at[pl.ds(r0, rows_tile)],
                            out_hbm.at[c, pl.ds(r0, rows_tile)])

        @pl.when(s == _NS - 1)
        def _():
            pltpu.sync_copy(acc.at[pl.ds(r0, rows_last)],
                            out_hbm.at[c, pl.ds(r0, rows_last)])

    return agg_kernel


_BM = 400


def _dot(a, b):
    return jnp.dot(a, b, preferred_element_type=jnp.float32)


def _layer1_tc(x, agg1, W1s, W1np, b1r, n_nodes, dh_in):
    grid = n_nodes // _BM

    def body(x_ref, a_ref, ws_ref, wn_ref, b_ref, h_ref, rdeg_ref):
        asum = a_ref[0] + a_ref[1]
        deg = asum[:, 128:129]
        r = 1.0 / jnp.maximum(deg, 1.0)
        neigh = _dot(asum, wn_ref[...]) * r
        h = jax.nn.relu(_dot(x_ref[...], ws_ref[...]) + neigh + b_ref[...])
        h_ref[0] = h[:, :128]
        h_ref[1] = h[:, 128:]
        rdeg_ref[...] = r

    return pl.pallas_call(
        body,
        grid=(grid,),
        in_specs=[
            pl.BlockSpec((_BM, dh_in), lambda m: (m, 0)),
            pl.BlockSpec((2, _BM, 144), lambda m: (0, m, 0)),
            pl.BlockSpec((dh_in, 256), lambda m: (0, 0)),
            pl.BlockSpec((144, 256), lambda m: (0, 0)),
            pl.BlockSpec((1, 256), lambda m: (0, 0)),
        ],
        out_specs=[
            pl.BlockSpec((2, _BM, 128), lambda m: (0, m, 0)),
            pl.BlockSpec((_BM, 1), lambda m: (m, 0)),
        ],
        out_shape=[
            jax.ShapeDtypeStruct((2, n_nodes, 128), jnp.float32),
            jax.ShapeDtypeStruct((n_nodes, 1), jnp.float32),
        ],
    )(x, agg1, W1s, W1np, b1r)


def _layer2_tc(h1s, agg2, rdeg, W2sa, W2sb, W2na, W2nb, b2r, W3np, n_nodes):
    grid = n_nodes // _BM

    def body(h_ref, a_ref, r_ref, wsa, wsb, wna, wnb, b_ref, w3n, h2_ref, t3_ref):
        r = r_ref[...]
        neigh = (_dot(a_ref[0], wna[...]) + _dot(a_ref[1], wnb[...])) * r
        h2 = jax.nn.relu(_dot(h_ref[0], wsa[...]) + _dot(h_ref[1], wsb[...])
                         + neigh + b_ref[...])
        t3_ref[...] = _dot(h2, w3n[...])
        h2_ref[0] = h2[:, :128]
        h2_ref[1] = h2[:, 128:]

    return pl.pallas_call(
        body,
        grid=(grid,),
        in_specs=[
            pl.BlockSpec((2, _BM, 128), lambda m: (0, m, 0)),
            pl.BlockSpec((2, _BM, 128), lambda m: (0, m, 0)),
            pl.BlockSpec((_BM, 1), lambda m: (m, 0)),
            pl.BlockSpec((128, 256), lambda m: (0, 0)),
            pl.BlockSpec((128, 256), lambda m: (0, 0)),
            pl.BlockSpec((128, 256), lambda m: (0, 0)),
            pl.BlockSpec((128, 256), lambda m: (0, 0)),
            pl.BlockSpec((1, 256), lambda m: (0, 0)),
            pl.BlockSpec((256, 64), lambda m: (0, 0)),
        ],
        out_specs=[
            pl.BlockSpec((2, _BM, 128), lambda m: (0, m, 0)),
            pl.BlockSpec((_BM, 64), lambda m: (m, 0)),
        ],
        out_shape=[
            jax.ShapeDtypeStruct((2, n_nodes, 128), jnp.float32),
            jax.ShapeDtypeStruct((n_nodes, 64), jnp.float32),
        ],
    )(h1s, agg2, rdeg, W2sa, W2sb, W2na, W2nb, b2r, W3np)


def _layer3_tc(h2s, agg3, rdeg, W3sa, W3sb, b3p, n_nodes):
    grid = n_nodes // _BM

    def body(h_ref, a_ref, r_ref, wsa, wsb, b_ref, o_ref):
        neigh = (a_ref[0] + a_ref[1]) * r_ref[...]
        o_ref[...] = (_dot(h_ref[0], wsa[...]) + _dot(h_ref[1], wsb[...])
                      + neigh + b_ref[...])

    return pl.pallas_call(
        body,
        grid=(grid,),
        in_specs=[
            pl.BlockSpec((2, _BM, 128), lambda m: (0, m, 0)),
            pl.BlockSpec((2, _BM, 64), lambda m: (0, m, 0)),
            pl.BlockSpec((_BM, 1), lambda m: (m, 0)),
            pl.BlockSpec((128, 64), lambda m: (0, 0)),
            pl.BlockSpec((128, 64), lambda m: (0, 0)),
            pl.BlockSpec((1, 64), lambda m: (0, 0)),
        ],
        out_specs=pl.BlockSpec((_BM, 64), lambda m: (m, 0)),
        out_shape=jax.ShapeDtypeStruct((n_nodes, 64), jnp.float32),
    )(h2s, agg3, rdeg, W3sa, W3sb, b3p)


def kernel(x, edge_index, W1_self, W1_neigh, b1, W2_self, W2_neigh, b2,
           W3_self, W3_neigh, b3):
    n_nodes, d_in = x.shape
    n_edges = edge_index.shape[1]
    d_out = W3_self.shape[1]

    unit = _NC * _NS * 2 * _K
    e_pad = -(-n_edges // unit) * unit
    npad = e_pad - n_edges
    pad_ar = jnp.arange(npad, dtype=jnp.int32)
    src = jnp.concatenate([edge_index[0], pad_ar % n_nodes])
    dst = jnp.concatenate([edge_index[1], n_nodes + pad_ar % _TRASH])

    ones = jnp.ones((n_nodes, 1), jnp.float32)
    zpad = jnp.zeros((n_nodes, 15), jnp.float32)
    xaug = jnp.concatenate([x, ones, zpad], axis=1)

    z144 = jnp.zeros((n_nodes, 144), jnp.float32)
    z128 = jnp.zeros((n_nodes, 128), jnp.float32)
    z64 = jnp.zeros((n_nodes, 64), jnp.float32)

    W1np = jnp.concatenate([W1_neigh, jnp.zeros((16, 256), jnp.float32)],
                           axis=0)
    b1r = b1.reshape(1, -1)
    W2sa, W2sb = W2_self[:128], W2_self[128:]
    W2na, W2nb = W2_neigh[:128], W2_neigh[128:]
    b2r = b2.reshape(1, -1)
    cpad = jnp.zeros((256, 64 - d_out), jnp.float32)
    W3np = jnp.concatenate([W3_neigh, cpad], axis=1)
    W3sp = jnp.concatenate([W3_self, cpad], axis=1)
    W3sa, W3sb = W3sp[:128], W3sp[128:]
    b3p = jnp.concatenate([b3, jnp.zeros((64 - d_out,), jnp.float32)]
                          ).reshape(1, -1)

    agg1 = _make_sc_agg(n_nodes, e_pad, 144, True)(xaug, src, dst, z144)
    h1s, rdeg = _layer1_tc(x, agg1, W1_self, W1np, b1r, n_nodes, d_in)
    agg2 = _make_sc_agg(n_nodes, e_pad, 128, False)(h1s, src, dst, z128)
    h2s, t3 = _layer2_tc(h1s, agg2, rdeg, W2sa, W2sb, W2na, W2nb, b2r,
                         W3np, n_nodes)
    agg3 = _make_sc_agg(n_nodes, e_pad, 64, True)(t3, src, dst, z64)
    outp = _layer3_tc(h2s, agg3, rdeg, W3sa, W3sb, b3p, n_nodes)
    return outp[:, :d_out]


# speedup vs baseline: 2.8716x; 1.2279x over previous
import functools

import jax
import jax.numpy as jnp
from jax import lax
from jax.experimental import pallas as pl
from jax.experimental.pallas import tpu as pltpu
from jax.experimental.pallas import tpu_sc as plsc

_NC = 2
_NS = 16
_K = 128
_TRASH = 256


def _make_sc_agg(n_nodes, e_pad, dh, edge_split, dtype=jnp.float32):
    n_workers = _NC * _NS if edge_split else _NS
    assert e_pad % (n_workers * 2 * _K) == 0
    trash = _TRASH
    e_w = e_pad // n_workers
    nch = e_w // _K
    rows_tile = -(-(n_nodes // _NS) // 8) * 8
    rows_last = n_nodes - (_NS - 1) * rows_tile
    assert 0 < rows_last <= rows_tile
    assert dh * jnp.dtype(dtype).itemsize % 64 == 0

    mesh = plsc.VectorSubcoreMesh(
        core_axis_name="c", subcore_axis_name="s",
        num_cores=_NC, num_subcores=_NS)

    assert nch % 2 == 0
    scratch = [
        pltpu.VMEM((_K,), jnp.int32), pltpu.VMEM((_K,), jnp.int32),
        pltpu.VMEM((_K,), jnp.int32), pltpu.VMEM((_K,), jnp.int32),
        pltpu.VMEM((_K, dh), dtype), pltpu.VMEM((_K, dh), dtype),
        pltpu.VMEM_SHARED((n_nodes + trash, dh), dtype),
        pltpu.SemaphoreType.DMA, pltpu.SemaphoreType.DMA,
        pltpu.SemaphoreType.DMA, pltpu.SemaphoreType.DMA,
        pltpu.SemaphoreType.DMA,
    ]

    @functools.partial(
        pl.kernel,
        out_type=jax.ShapeDtypeStruct((_NC, n_nodes, dh), dtype),
        mesh=mesh,
        scratch_types=scratch,
        compiler_params=pltpu.CompilerParams(use_tc_tiling_on_sc=False),
    )
    def agg_kernel(t_hbm, src_hbm, dst_hbm, z_hbm, out_hbm,
                   si0, si1, di0, di1, rw0, rw1, acc, sg0, sg1, ss0, ss1,
                   sz):
        c = lax.axis_index("c")
        s = lax.axis_index("s")
        if edge_split:
            tbl = t_hbm
            base = (c * _NS + s) * e_w
        else:
            tbl = t_hbm.at[c]
            base = s * e_w

        r0 = s * rows_tile

        @pl.when(s < _NS - 1)
        def _():
            pltpu.async_copy(z_hbm, acc.at[pl.ds(r0, rows_tile)], sz)

        @pl.when(s == _NS - 1)
        def _():
            pltpu.async_copy(z_hbm.at[pl.ds(0, rows_last)],
                             acc.at[pl.ds(r0, rows_last)], sz)

        slabs = ((si0, di0, rw0, sg0, ss0), (si1, di1, rw1, sg1, ss1))

        def iter_body(ii, p):
            si, di, rw, sg, ss = slabs[p]
            qsi, qdi, qrw, qsg, qss = slabs[1 - p]

            @pl.when(ii >= 2)
            def _():
                pltpu.make_async_copy(rw, acc.at[di], ss).wait()

            @pl.when(ii < nch)
            def _():
                off = base + ii * _K
                pltpu.sync_copy(src_hbm.at[pl.ds(off, _K)], si)
                pltpu.sync_copy(dst_hbm.at[pl.ds(off, _K)], di)
                pltpu.async_copy(tbl.at[si], rw, sg)

            @pl.when(ii == 1)
            def _():
                @pl.when(s < _NS - 1)
                def _():
                    pltpu.make_async_copy(
                        z_hbm, acc.at[pl.ds(r0, rows_tile)], sz).wait()

                @pl.when(s == _NS - 1)
                def _():
                    pltpu.make_async_copy(
                        z_hbm.at[pl.ds(0, rows_last)],
                        acc.at[pl.ds(r0, rows_last)], sz).wait()

                plsc.subcore_barrier()

            @pl.when((ii >= 1) & (ii <= nch))
            def _():
                pltpu.make_async_copy(tbl.at[qsi], qrw, qsg).wait()
                pltpu.async_copy(qrw, acc.at[qdi], qss, add=True)

        @pl.loop(0, nch + 2, step=2)
        def _(ii):
            iter_body(ii, 0)
            iter_body(ii + 1, 1)

        plsc.subcore_barrier()

        @pl.when(s < _NS - 1)
        def _():
            pltpu.sync_copy(acc.at[pl.ds(r0, rows_tile)],
                            out_hbm.at[c, pl.ds(r0, rows_tile)])

        @pl.when(s == _NS - 1)
        def _():
            pltpu.sync_copy(acc.at[pl.ds(r0, rows_last)],
                            out_hbm.at[c, pl.ds(r0, rows_last)])

    return agg_kernel


_BM = 2000


def _dot(a, b):
    return jnp.dot(a, b, preferred_element_type=jnp.float32)


def _layer1_tc(x, agg1, W1s, W1np, b1r, n_nodes, dh_in):
    grid = n_nodes // _BM

    def body(x_ref, a_ref, ws_ref, wn_ref, b_ref, h_ref, hb_ref, rdeg_ref):
        asum = (a_ref[0].astype(jnp.float32)
                + a_ref[1].astype(jnp.float32))
        deg = asum[:, 128:129]
        r = 1.0 / jnp.maximum(deg, 1.0)
        neigh = _dot(asum, wn_ref[...]) * r
        h = jax.nn.relu(_dot(x_ref[...], ws_ref[...]) + neigh + b_ref[...])
        h_ref[0] = h[:, :128]
        h_ref[1] = h[:, 128:]
        hb_ref[...] = h.astype(jnp.bfloat16)
        rdeg_ref[...] = r

    return pl.pallas_call(
        body,
        grid=(grid,),
        in_specs=[
            pl.BlockSpec((_BM, dh_in), lambda m: (m, 0)),
            pl.BlockSpec((2, _BM, 160), lambda m: (0, m, 0)),
            pl.BlockSpec((dh_in, 256), lambda m: (0, 0)),
            pl.BlockSpec((160, 256), lambda m: (0, 0)),
            pl.BlockSpec((1, 256), lambda m: (0, 0)),
        ],
        out_specs=[
            pl.BlockSpec((2, _BM, 128), lambda m: (0, m, 0)),
            pl.BlockSpec((_BM, 256), lambda m: (m, 0)),
            pl.BlockSpec((_BM, 1), lambda m: (m, 0)),
        ],
        out_shape=[
            jax.ShapeDtypeStruct((2, n_nodes, 128), jnp.float32),
            jax.ShapeDtypeStruct((n_nodes, 256), jnp.bfloat16),
            jax.ShapeDtypeStruct((n_nodes, 1), jnp.float32),
        ],
    )(x, agg1, W1s, W1np, b1r)


def _layer2_tc(h1s, agg2, rdeg, W2sa, W2sb, W2n, b2r, W3np, n_nodes):
    grid = n_nodes // _BM

    def body(h_ref, a_ref, r_ref, wsa, wsb, wn, b_ref, w3n, h2_ref, t3_ref):
        r = r_ref[...]
        asum = (a_ref[0].astype(jnp.float32)
                + a_ref[1].astype(jnp.float32))
        neigh = _dot(asum, wn[...]) * r
        h2 = jax.nn.relu(_dot(h_ref[0], wsa[...]) + _dot(h_ref[1], wsb[...])
                         + neigh + b_ref[...])
        t3_ref[...] = _dot(h2, w3n[...])
        h2_ref[0] = h2[:, :128]
        h2_ref[1] = h2[:, 128:]

    return pl.pallas_call(
        body,
        grid=(grid,),
        in_specs=[
            pl.BlockSpec((2, _BM, 128), lambda m: (0, m, 0)),
            pl.BlockSpec((2, _BM, 256), lambda m: (0, m, 0)),
            pl.BlockSpec((_BM, 1), lambda m: (m, 0)),
            pl.BlockSpec((128, 256), lambda m: (0, 0)),
            pl.BlockSpec((128, 256), lambda m: (0, 0)),
            pl.BlockSpec((256, 256), lambda m: (0, 0)),
            pl.BlockSpec((1, 256), lambda m: (0, 0)),
            pl.BlockSpec((256, 48), lambda m: (0, 0)),
        ],
        out_specs=[
            pl.BlockSpec((2, _BM, 128), lambda m: (0, m, 0)),
            pl.BlockSpec((_BM, 48), lambda m: (m, 0)),
        ],
        out_shape=[
            jax.ShapeDtypeStruct((2, n_nodes, 128), jnp.float32),
            jax.ShapeDtypeStruct((n_nodes, 48), jnp.float32),
        ],
    )(h1s, agg2, rdeg, W2sa, W2sb, W2n, b2r, W3np)


def _layer3_tc(h2s, agg3, rdeg, W3sa, W3sb, b3p, n_nodes):
    grid = n_nodes // _BM

    def body(h_ref, a_ref, r_ref, wsa, wsb, b_ref, o_ref):
        neigh = (a_ref[0] + a_ref[1]) * r_ref[...]
        o_ref[...] = (_dot(h_ref[0], wsa[...]) + _dot(h_ref[1], wsb[...])
                      + neigh + b_ref[...])

    return pl.pallas_call(
        body,
        grid=(grid,),
        in_specs=[
            pl.BlockSpec((2, _BM, 128), lambda m: (0, m, 0)),
            pl.BlockSpec((2, _BM, 48), lambda m: (0, m, 0)),
            pl.BlockSpec((_BM, 1), lambda m: (m, 0)),
            pl.BlockSpec((128, 48), lambda m: (0, 0)),
            pl.BlockSpec((128, 48), lambda m: (0, 0)),
            pl.BlockSpec((1, 48), lambda m: (0, 0)),
        ],
        out_specs=pl.BlockSpec((_BM, 48), lambda m: (m, 0)),
        out_shape=jax.ShapeDtypeStruct((n_nodes, 48), jnp.float32),
    )(h2s, agg3, rdeg, W3sa, W3sb, b3p)


def kernel(x, edge_index, W1_self, W1_neigh, b1, W2_self, W2_neigh, b2,
           W3_self, W3_neigh, b3):
    n_nodes, d_in = x.shape
    n_edges = edge_index.shape[1]
    d_out = W3_self.shape[1]

    unit = _NC * _NS * 2 * _K
    e_pad = -(-n_edges // unit) * unit
    npad = e_pad - n_edges
    pad_ar = jnp.arange(npad, dtype=jnp.int32)
    src = jnp.concatenate([edge_index[0], pad_ar % n_nodes])
    dst = jnp.concatenate([edge_index[1], n_nodes + pad_ar % _TRASH])

    ones = jnp.ones((n_nodes, 1), jnp.float32)
    zpad = jnp.zeros((n_nodes, 31), jnp.float32)
    xaug = jnp.concatenate([x, ones, zpad], axis=1
                           ).astype(jnp.bfloat16)

    rt = -(-(n_nodes // _NS) // 8) * 8
    z160 = jnp.zeros((rt, 160), jnp.bfloat16)
    z256 = jnp.zeros((rt, 256), jnp.bfloat16)
    z48 = jnp.zeros((rt, 48), jnp.float32)

    W1np = jnp.concatenate([W1_neigh, jnp.zeros((32, 256), jnp.float32)],
                           axis=0)
    b1r = b1.reshape(1, -1)
    W2sa, W2sb = W2_self[:128], W2_self[128:]
    b2r = b2.reshape(1, -1)
    cpad = jnp.zeros((256, 48 - d_out), jnp.float32)
    W3np = jnp.concatenate([W3_neigh, cpad], axis=1)
    W3sp = jnp.concatenate([W3_self, cpad], axis=1)
    W3sa, W3sb = W3sp[:128], W3sp[128:]
    b3p = jnp.concatenate([b3, jnp.zeros((48 - d_out,), jnp.float32)]
                          ).reshape(1, -1)

    agg1 = _make_sc_agg(n_nodes, e_pad, 160, True, jnp.bfloat16)(
        xaug, src, dst, z160)
    h1s, h1b, rdeg = _layer1_tc(x, agg1, W1_self, W1np, b1r, n_nodes, d_in)
    agg2 = _make_sc_agg(n_nodes, e_pad, 256, True, jnp.bfloat16)(
        h1b, src, dst, z256)
    h2s, t3 = _layer2_tc(h1s, agg2, rdeg, W2sa, W2sb, W2_neigh, b2r,
                         W3np, n_nodes)
    agg3 = _make_sc_agg(n_nodes, e_pad, 48, True)(t3, src, dst, z48)
    outp = _layer3_tc(h2s, agg3, rdeg, W3sa, W3sb, b3p, n_nodes)
    return outp[:, :d_out]
